# Initial kernel scaffold; baseline (speedup 1.0000x reference)
import jax
import jax.numpy as jnp
from jax.experimental import pallas as pl


def kernel(uv_maps, position_map):
    raise NotImplementedError("write your pallas kernel here")


# speedup vs baseline: 8.0461x; 8.0461x over previous
import functools

import jax
import jax.numpy as jnp
from jax import lax
from jax.experimental import pallas as pl
from jax.experimental.pallas import tpu as pltpu
from jax.experimental.pallas import tpu_sc as plsc

IMG = 512
NPIX = IMG * IMG
NB = 4
NPTS = 65536
NC, NS, VL = 2, 16, 16
PPT = NPTS // NS
BPC = NB // NC
SH_WORDS = 3 * NPIX
TILE_WORDS = SH_WORDS // NS
CHUNK = 12288
OUT_WORDS = NB * 3 * NPIX


def _body(uv_hbm, pos_hbm, out_hbm, ch_v, idx_v, val_v, obuf_v, acc_sh, sem):
    c = lax.axis_index("c")
    s = lax.axis_index("s")
    sh_base = s * TILE_WORDS

    for k in range(BPC):
        b = BPC * c + k

        zeros = jnp.zeros((VL,), jnp.float32)

        def zero_body(i, carry):
            obuf_v[pl.ds(i * VL, VL)] = zeros
            return carry

        lax.fori_loop(0, CHUNK // VL, zero_body, 0)
        for j in range(TILE_WORDS // CHUNK):
            pltpu.sync_copy(obuf_v,
                            acc_sh.at[pl.ds(sh_base + j * CHUNK, CHUNK)])
        plsc.subcore_barrier()

        srcs = ((uv_hbm, 0), (uv_hbm, 1), (uv_hbm, 10), (uv_hbm, 11),
                (uv_hbm, 12), (uv_hbm, 13), (pos_hbm, 0), (pos_hbm, 1))
        copies = [
            pltpu.async_copy(ref.at[b, ch, pl.ds(s * PPT, PPT)],
                             ch_v.at[ci], sem)
            for ci, (ref, ch) in enumerate(srcs)
        ]
        for cp in copies:
            cp.wait()

        def comp_body(i, carry):
            sl = pl.ds(i * VL, VL)
            xf = (ch_v[6, sl] + ch_v[0, sl] + 1.0) * 256.0
            yf = (ch_v[7, sl] + ch_v[1, sl] + 1.0) * 256.0
            xi = jnp.clip(xf, 0.0, 511.0).astype(jnp.int32)
            yi = jnp.clip(yf, 0.0, 511.0).astype(jnp.int32)
            pix = yi * IMG + xi
            opac = 1.0 / (1.0 + jnp.exp(-ch_v[2, sl]))
            idx_v[sl] = pix
            idx_v[pl.ds(PPT + i * VL, VL)] = pix + NPIX
            idx_v[pl.ds(2 * PPT + i * VL, VL)] = pix + 2 * NPIX
            val_v[sl] = ch_v[3, sl] * opac
            val_v[pl.ds(PPT + i * VL, VL)] = ch_v[4, sl] * opac
            val_v[pl.ds(2 * PPT + i * VL, VL)] = ch_v[5, sl] * opac
            return carry

        lax.fori_loop(0, PPT // VL, comp_body, 0)
        pltpu.sync_copy(val_v, acc_sh.at[idx_v], add=True)

        plsc.subcore_barrier()

        out_base = b * SH_WORDS + sh_base
        for j in range(TILE_WORDS // CHUNK):
            pltpu.sync_copy(acc_sh.at[pl.ds(sh_base + j * CHUNK, CHUNK)],
                            obuf_v)

            def clip_body(i, carry):
                sl = pl.ds(i * VL, VL)
                obuf_v[sl] = jnp.clip(obuf_v[sl], 0.0, 1.0)
                return carry

            lax.fori_loop(0, CHUNK // VL, clip_body, 0)
            pltpu.sync_copy(obuf_v,
                            out_hbm.at[pl.ds(out_base + j * CHUNK, CHUNK)])

        plsc.subcore_barrier()


_render = pl.kernel(
    _body,
    out_type=jax.ShapeDtypeStruct((OUT_WORDS,), jnp.float32),
    mesh=plsc.VectorSubcoreMesh(core_axis_name="c", subcore_axis_name="s"),
    scratch_types=[
        pltpu.VMEM((8, PPT), jnp.float32),
        pltpu.VMEM((3 * PPT,), jnp.int32),
        pltpu.VMEM((3 * PPT,), jnp.float32),
        pltpu.VMEM((CHUNK,), jnp.float32),
        pltpu.VMEM_SHARED((SH_WORDS,), jnp.float32),
        pltpu.SemaphoreType.DMA,
    ],
)


def kernel(uv_maps, position_map):
    uv = uv_maps.reshape(NB, 14, NPTS)
    pos = position_map.reshape(NB, 3, NPTS)
    out = _render(uv, pos)
    return out.reshape(NB, 3, IMG, IMG)
